# baseline (speedup 1.0000x reference)
import functools

import jax
import jax.numpy as jnp
from jax import lax
from jax.experimental import pallas as pl
from jax.experimental.pallas import tpu as pltpu
from jax.experimental.pallas import tpu_sc as plsc

NC = 2
NS = 16
NW = NC * NS
D = 128
N_REAL = 10000
NP = 10240
ROWS_PER_TILE = NP // NS
CHUNK = 128
SEC = 16
NSEC = 5
CPT = SEC * NSEC
E_PAD = NW * CPT * CHUNK
N_SCRATCH = NP - N_REAL
BLK = 1024


HR = NP // D


def _sc_deg_body(dst_hbm, zeros_hbm, iota_hbm, out_hbm,
                 dst_v, hist_v, iota_v, acc):
  c = lax.axis_index("c")
  s = lax.axis_index("s")
  w = c * NS + s
  pltpu.sync_copy(dst_hbm.at[w], dst_v)
  pltpu.sync_copy(zeros_hbm, hist_v)
  pltpu.sync_copy(iota_hbm, iota_v)

  @pl.when(s == 0)
  def _():
    pltpu.sync_copy(hist_v, acc)
  ones16 = jnp.ones((16,), jnp.float32)

  def row(j, carry):
    for k in range(CHUNK // 16):
      d16 = dst_v[j, pl.ds(16 * k, 16)]
      plsc.addupdate_scatter(
          hist_v, [lax.shift_right_logical(d16, 7), lax.bitwise_and(d16, 127)],
          ones16)
    return carry

  lax.fori_loop(0, CPT, row, 0)
  plsc.subcore_barrier()
  pltpu.sync_copy(hist_v, acc.at[iota_v], add=True)
  plsc.subcore_barrier()
  @pl.when(s < HR // 8)
  def _():
    pltpu.sync_copy(acc.at[pl.ds(s * 8, 8)], hist_v.at[pl.ds(0, 8)])
    pltpu.sync_copy(hist_v.at[pl.ds(0, 8)], out_hbm.at[c, pl.ds(s * 8, 8)])


def _sc_scatter_body(u_hbm, src_hbm, dst_hbm, zeros_hbm, out_hbm,
                     srcA, srcB, dstA, dstB, buf0, buf1,
                     sem0, sem1, semS0, semS1, semi, acc):
  c = lax.axis_index("c")
  s = lax.axis_index("s")
  w = c * NS + s
  row0 = s * ROWS_PER_TILE
  pltpu.sync_copy(src_hbm.at[w, pl.ds(0, SEC)], srcA)
  pltpu.sync_copy(dst_hbm.at[w, pl.ds(0, SEC)], dstA)
  pltpu.sync_copy(zeros_hbm, buf0)
  for z in range(ROWS_PER_TILE // CHUNK):
    pltpu.sync_copy(buf0, acc.at[pl.ds(row0 + z * CHUNK, CHUNK)])
  plsc.subcore_barrier()

  secs = [(srcA, dstA), (srcB, dstB)]
  pltpu.async_copy(u_hbm.at[srcA.at[0]], buf0, sem0)
  pltpu.async_copy(u_hbm.at[srcA.at[1]], buf1, sem1)
  for sct in range(NSEC):
    src_v, dst_v = secs[sct % 2]
    nsrc_v, ndst_v = secs[(sct + 1) % 2]
    if sct + 1 < NSEC:
      ip0 = pltpu.async_copy(src_hbm.at[w, pl.ds((sct + 1) * SEC, SEC)],
                             nsrc_v, semi)
      ip1 = pltpu.async_copy(dst_hbm.at[w, pl.ds((sct + 1) * SEC, SEC)],
                             ndst_v, semi)

    def pair(j, carry, src_v=src_v, dst_v=dst_v):
      c0 = 2 * j
      pltpu.make_async_copy(u_hbm.at[src_v.at[c0]], buf0, sem0).wait()
      pltpu.async_copy(buf0, acc.at[dst_v.at[c0]], semS0, add=True)
      pltpu.make_async_copy(u_hbm.at[src_v.at[c0 + 1]], buf1, sem1).wait()
      pltpu.async_copy(buf1, acc.at[dst_v.at[c0 + 1]], semS1, add=True)
      pltpu.make_async_copy(buf0, acc.at[dst_v.at[c0]], semS0).wait()

      @pl.when(c0 + 2 < SEC)
      def _():
        pltpu.async_copy(u_hbm.at[src_v.at[c0 + 2]], buf0, sem0)

      pltpu.make_async_copy(buf1, acc.at[dst_v.at[c0 + 1]], semS1).wait()

      @pl.when(c0 + 3 < SEC)
      def _():
        pltpu.async_copy(u_hbm.at[src_v.at[c0 + 3]], buf1, sem1)

      return carry

    lax.fori_loop(0, SEC // 2, pair, 0)
    if sct + 1 < NSEC:
      ip0.wait()
      ip1.wait()
      pltpu.async_copy(u_hbm.at[nsrc_v.at[0]], buf0, sem0)
      pltpu.async_copy(u_hbm.at[nsrc_v.at[1]], buf1, sem1)

  plsc.subcore_barrier()
  for z in range(ROWS_PER_TILE // CHUNK):
    r = row0 + z * CHUNK
    pltpu.sync_copy(acc.at[pl.ds(r, CHUNK)], buf0)
    pltpu.sync_copy(buf0, out_hbm.at[c, pl.ds(r, CHUNK)])


def _dis_block(p0b, p1b):
  deg = p0b + p1b + 1.0
  return lax.rsqrt(deg)


def _tc1_body(xb, w1, p0b, p1b, ub):
  z = jnp.dot(xb[...], w1[...], preferred_element_type=jnp.float32)
  ub[...] = z * _dis_block(p0b[...], p1b[...])


def _tc2_body(q0b, q1b, u1b, w2, b1r, p0b, p1b, ub):
  dis = _dis_block(p0b[...], p1b[...])
  agg = (q0b[...] + q1b[...] + u1b[...]) * dis
  h = jnp.maximum(agg + b1r[...], 0.0)
  ub[...] = jnp.dot(h, w2[...], preferred_element_type=jnp.float32) * dis


def _tc3_body(q0b, q1b, u2b, b2r, p0b, p1b, ob):
  dis = _dis_block(p0b[...], p1b[...])
  ob[...] = (q0b[...] + q1b[...] + u2b[...]) * dis + b2r[...]


def _node_specs(*widths):
  return [pl.BlockSpec((BLK, wd), lambda i: (i, 0)) for wd in widths]


def _full_spec(shape):
  return pl.BlockSpec(shape, lambda i: (0,) * len(shape))


@functools.lru_cache(maxsize=None)
def _sc_kernels():
  mesh = plsc.VectorSubcoreMesh(
      core_axis_name="c", subcore_axis_name="s", num_cores=NC, num_subcores=NS)
  sc_deg = pl.kernel(
      _sc_deg_body,
      out_type=jax.ShapeDtypeStruct((NC, HR, D), jnp.float32),
      mesh=mesh,
      scratch_types=[
          pltpu.VMEM((CPT, CHUNK), jnp.int32),
          pltpu.VMEM((HR, D), jnp.float32),
          pltpu.VMEM((HR,), jnp.int32),
          pltpu.VMEM_SHARED((HR, D), jnp.float32),
      ],
      compiler_params=pltpu.CompilerParams(needs_layout_passes=False),
  )
  sc_scatter = pl.kernel(
      _sc_scatter_body,
      out_type=jax.ShapeDtypeStruct((NC, NP, D), jnp.float32),
      mesh=mesh,
      scratch_types=[
          pltpu.VMEM((SEC, CHUNK), jnp.int32),
          pltpu.VMEM((SEC, CHUNK), jnp.int32),
          pltpu.VMEM((SEC, CHUNK), jnp.int32),
          pltpu.VMEM((SEC, CHUNK), jnp.int32),
          pltpu.VMEM((CHUNK, D), jnp.float32),
          pltpu.VMEM((CHUNK, D), jnp.float32),
          pltpu.SemaphoreType.DMA,
          pltpu.SemaphoreType.DMA,
          pltpu.SemaphoreType.DMA,
          pltpu.SemaphoreType.DMA,
          pltpu.SemaphoreType.DMA,
          pltpu.VMEM_SHARED((NP, D), jnp.float32),
      ],
  )
  return sc_deg, sc_scatter


_GRID = (NP // BLK,)

_tc1 = pl.pallas_call(
    _tc1_body,
    grid=_GRID,
    in_specs=[*_node_specs(D), _full_spec((D, D)), *_node_specs(1, 1)],
    out_specs=_node_specs(D)[0],
    out_shape=jax.ShapeDtypeStruct((NP, D), jnp.float32),
)

_tc2 = pl.pallas_call(
    _tc2_body,
    grid=_GRID,
    in_specs=[*_node_specs(D, D, D), _full_spec((D, D)), _full_spec((1, D)),
              *_node_specs(1, 1)],
    out_specs=_node_specs(D)[0],
    out_shape=jax.ShapeDtypeStruct((NP, D), jnp.float32),
)

_tc3 = pl.pallas_call(
    _tc3_body,
    grid=_GRID,
    in_specs=[*_node_specs(D, D, D), _full_spec((1, D)),
              *_node_specs(1, 1)],
    out_specs=_node_specs(D)[0],
    out_shape=jax.ShapeDtypeStruct((N_REAL, D), jnp.float32),
)


@jax.jit
def kernel(x, edge_index, W1, b1, W2, b2):
  n_edges = edge_index.shape[1]
  src = edge_index[0].astype(jnp.int32)
  dst = edge_index[1].astype(jnp.int32)
  pad_n = E_PAD - n_edges
  pad_idx = N_REAL + (jnp.arange(pad_n, dtype=jnp.int32) % N_SCRATCH)
  src_p = jnp.concatenate([src, pad_idx]).reshape(NW, CPT, CHUNK)
  dst_p = jnp.concatenate([dst, pad_idx]).reshape(NW, CPT, CHUNK)

  zeros80 = jnp.zeros((HR, D), jnp.float32)
  iota80 = jnp.arange(HR, dtype=jnp.int32)
  zeros128 = jnp.zeros((CHUNK, D), jnp.float32)

  sc_deg, sc_scatter = _sc_kernels()
  degp = sc_deg(dst_p, zeros80, iota80)
  p0 = degp[0].reshape(NP, 1)
  p1 = degp[1].reshape(NP, 1)

  u1 = _tc1(x, W1, p0, p1)
  q = sc_scatter(u1, src_p, dst_p, zeros128)
  u2 = _tc2(q[0], q[1], u1, W2, b1.reshape(1, D), p0, p1)
  q2 = sc_scatter(u2, src_p, dst_p, zeros128)
  return _tc3(q2[0], q2[1], u2, b2.reshape(1, D), p0, p1)


# speedup vs baseline: 1.1919x; 1.1919x over previous
import functools

import jax
import jax.numpy as jnp
from jax import lax
from jax.experimental import pallas as pl
from jax.experimental.pallas import tpu as pltpu
from jax.experimental.pallas import tpu_sc as plsc

NC = 2
NS = 16
NW = NC * NS
D = 128
N_REAL = 10000
NP = 10240
ROWS_PER_TILE = NP // NS
CHUNK = 128
SEC = 16
NSEC = 5
CPT = SEC * NSEC
E_PAD = NW * CPT * CHUNK
N_SCRATCH = NP - N_REAL
BLK = 1024


HR = NP // D


def _sc_deg_body(dst_hbm, zeros_hbm, iota_hbm, out_hbm,
                 dst_v, hist_v, iota_v, acc):
  c = lax.axis_index("c")
  s = lax.axis_index("s")
  w = c * NS + s
  pltpu.sync_copy(dst_hbm.at[w], dst_v)
  pltpu.sync_copy(zeros_hbm, hist_v)
  pltpu.sync_copy(iota_hbm, iota_v)

  @pl.when(s == 0)
  def _():
    pltpu.sync_copy(hist_v, acc)
  ones16 = jnp.ones((16,), jnp.float32)

  def row(j, carry):
    for k in range(CHUNK // 16):
      d16 = dst_v[j, pl.ds(16 * k, 16)]
      plsc.addupdate_scatter(
          hist_v, [lax.shift_right_logical(d16, 7), lax.bitwise_and(d16, 127)],
          ones16)
    return carry

  lax.fori_loop(0, CPT, row, 0)
  plsc.subcore_barrier()
  pltpu.sync_copy(hist_v, acc.at[iota_v], add=True)
  plsc.subcore_barrier()
  @pl.when(s < HR // 8)
  def _():
    pltpu.sync_copy(acc.at[pl.ds(s * 8, 8)], hist_v.at[pl.ds(0, 8)])
    pltpu.sync_copy(hist_v.at[pl.ds(0, 8)], out_hbm.at[c, pl.ds(s * 8, 8)])


def _sc_scatter_body(u_hbm, src_hbm, dst_hbm, zeros_hbm, out_hbm,
                     srcA, srcB, dstA, dstB, buf0, buf1,
                     sem0, sem1, semi, acc):
  c = lax.axis_index("c")
  s = lax.axis_index("s")
  w = c * NS + s
  row0 = s * ROWS_PER_TILE
  pltpu.sync_copy(src_hbm.at[w, pl.ds(0, SEC)], srcA)
  pltpu.sync_copy(dst_hbm.at[w, pl.ds(0, SEC)], dstA)
  pltpu.sync_copy(zeros_hbm, buf0)
  for z in range(ROWS_PER_TILE // CHUNK):
    pltpu.sync_copy(buf0, acc.at[pl.ds(row0 + z * CHUNK, CHUNK)])
  plsc.subcore_barrier()

  secs = [(srcA, dstA), (srcB, dstB)]
  pltpu.async_copy(u_hbm.at[srcA.at[0]], buf0, sem0)
  for sct in range(NSEC):
    src_v, dst_v = secs[sct % 2]
    nsrc_v, ndst_v = secs[(sct + 1) % 2]
    if sct + 1 < NSEC:
      ip0 = pltpu.async_copy(src_hbm.at[w, pl.ds((sct + 1) * SEC, SEC)],
                             nsrc_v, semi)
      ip1 = pltpu.async_copy(dst_hbm.at[w, pl.ds((sct + 1) * SEC, SEC)],
                             ndst_v, semi)

    def pair(j, carry, src_v=src_v, dst_v=dst_v):
      c0 = 2 * j
      pltpu.async_copy(u_hbm.at[src_v.at[c0 + 1]], buf1, sem1)
      pltpu.make_async_copy(u_hbm.at[src_v.at[c0]], buf0, sem0).wait()
      pltpu.sync_copy(buf0, acc.at[dst_v.at[c0]], add=True)

      @pl.when(j + 1 < SEC // 2)
      def _():
        pltpu.async_copy(u_hbm.at[src_v.at[c0 + 2]], buf0, sem0)

      pltpu.make_async_copy(u_hbm.at[src_v.at[c0 + 1]], buf1, sem1).wait()
      pltpu.sync_copy(buf1, acc.at[dst_v.at[c0 + 1]], add=True)
      return carry

    lax.fori_loop(0, SEC // 2, pair, 0)
    if sct + 1 < NSEC:
      ip0.wait()
      ip1.wait()
      pltpu.async_copy(u_hbm.at[nsrc_v.at[0]], buf0, sem0)

  plsc.subcore_barrier()
  for z in range(ROWS_PER_TILE // CHUNK):
    r = row0 + z * CHUNK
    pltpu.sync_copy(acc.at[pl.ds(r, CHUNK)], buf0)
    pltpu.sync_copy(buf0, out_hbm.at[c, pl.ds(r, CHUNK)])


def _dis_block(p0b, p1b):
  deg = p0b + p1b + 1.0
  return lax.rsqrt(deg)


def _tc1_body(xb, w1, p0b, p1b, ub):
  z = jnp.dot(xb[...], w1[...], preferred_element_type=jnp.float32)
  ub[...] = z * _dis_block(p0b[...], p1b[...])


def _tc2_body(q0b, q1b, u1b, w2, b1r, p0b, p1b, ub):
  dis = _dis_block(p0b[...], p1b[...])
  agg = (q0b[...] + q1b[...] + u1b[...]) * dis
  h = jnp.maximum(agg + b1r[...], 0.0)
  ub[...] = jnp.dot(h, w2[...], preferred_element_type=jnp.float32) * dis


def _tc3_body(q0b, q1b, u2b, b2r, p0b, p1b, ob):
  dis = _dis_block(p0b[...], p1b[...])
  ob[...] = (q0b[...] + q1b[...] + u2b[...]) * dis + b2r[...]


def _node_specs(*widths):
  return [pl.BlockSpec((BLK, wd), lambda i: (i, 0)) for wd in widths]


def _full_spec(shape):
  return pl.BlockSpec(shape, lambda i: (0,) * len(shape))


@functools.lru_cache(maxsize=None)
def _sc_kernels():
  mesh = plsc.VectorSubcoreMesh(
      core_axis_name="c", subcore_axis_name="s", num_cores=NC, num_subcores=NS)
  sc_deg = pl.kernel(
      _sc_deg_body,
      out_type=jax.ShapeDtypeStruct((NC, HR, D), jnp.float32),
      mesh=mesh,
      scratch_types=[
          pltpu.VMEM((CPT, CHUNK), jnp.int32),
          pltpu.VMEM((HR, D), jnp.float32),
          pltpu.VMEM((HR,), jnp.int32),
          pltpu.VMEM_SHARED((HR, D), jnp.float32),
      ],
      compiler_params=pltpu.CompilerParams(needs_layout_passes=False),
  )
  sc_scatter = pl.kernel(
      _sc_scatter_body,
      out_type=jax.ShapeDtypeStruct((NC, NP, D), jnp.float32),
      mesh=mesh,
      scratch_types=[
          pltpu.VMEM((SEC, CHUNK), jnp.int32),
          pltpu.VMEM((SEC, CHUNK), jnp.int32),
          pltpu.VMEM((SEC, CHUNK), jnp.int32),
          pltpu.VMEM((SEC, CHUNK), jnp.int32),
          pltpu.VMEM((CHUNK, D), jnp.float32),
          pltpu.VMEM((CHUNK, D), jnp.float32),
          pltpu.SemaphoreType.DMA,
          pltpu.SemaphoreType.DMA,
          pltpu.SemaphoreType.DMA,
          pltpu.VMEM_SHARED((NP, D), jnp.float32),
      ],
  )
  return sc_deg, sc_scatter


_GRID = (NP // BLK,)

_tc1 = pl.pallas_call(
    _tc1_body,
    grid=_GRID,
    in_specs=[*_node_specs(D), _full_spec((D, D)), *_node_specs(1, 1)],
    out_specs=_node_specs(D)[0],
    out_shape=jax.ShapeDtypeStruct((NP, D), jnp.float32),
)

_tc2 = pl.pallas_call(
    _tc2_body,
    grid=_GRID,
    in_specs=[*_node_specs(D, D, D), _full_spec((D, D)), _full_spec((1, D)),
              *_node_specs(1, 1)],
    out_specs=_node_specs(D)[0],
    out_shape=jax.ShapeDtypeStruct((NP, D), jnp.float32),
)

_tc3 = pl.pallas_call(
    _tc3_body,
    grid=_GRID,
    in_specs=[*_node_specs(D, D, D), _full_spec((1, D)),
              *_node_specs(1, 1)],
    out_specs=_node_specs(D)[0],
    out_shape=jax.ShapeDtypeStruct((N_REAL, D), jnp.float32),
)


@jax.jit
def kernel(x, edge_index, W1, b1, W2, b2):
  n_edges = edge_index.shape[1]
  src = edge_index[0].astype(jnp.int32)
  dst = edge_index[1].astype(jnp.int32)
  pad_n = E_PAD - n_edges
  pad_idx = N_REAL + (jnp.arange(pad_n, dtype=jnp.int32) % N_SCRATCH)
  src_p = jnp.concatenate([src, pad_idx]).reshape(NW, CPT, CHUNK)
  dst_p = jnp.concatenate([dst, pad_idx]).reshape(NW, CPT, CHUNK)

  zeros80 = jnp.zeros((HR, D), jnp.float32)
  iota80 = jnp.arange(HR, dtype=jnp.int32)
  zeros128 = jnp.zeros((CHUNK, D), jnp.float32)

  sc_deg, sc_scatter = _sc_kernels()
  degp = sc_deg(dst_p, zeros80, iota80)
  p0 = degp[0].reshape(NP, 1)
  p1 = degp[1].reshape(NP, 1)

  u1 = _tc1(x, W1, p0, p1)
  q = sc_scatter(u1, src_p, dst_p, zeros128)
  u2 = _tc2(q[0], q[1], u1, W2, b1.reshape(1, D), p0, p1)
  q2 = sc_scatter(u2, src_p, dst_p, zeros128)
  return _tc3(q2[0], q2[1], u2, b2.reshape(1, D), p0, p1)


# speedup vs baseline: 1.2811x; 1.0749x over previous
import functools

import jax
import jax.numpy as jnp
from jax import lax
from jax.experimental import pallas as pl
from jax.experimental.pallas import tpu as pltpu
from jax.experimental.pallas import tpu_sc as plsc

NC = 2
NS = 16
NW = NC * NS
D = 128
N_REAL = 10000
NP = 10240
ROWS_PER_TILE = NP // NS
CHUNK = 128
SEC = 16
NSEC = 5
CPT = SEC * NSEC
E_PAD = NW * CPT * CHUNK
N_SCRATCH = NP - N_REAL
BLK = 1024


HR = NP // D


def _sc_deg_body(dst_hbm, zeros_hbm, iota_hbm, out0_hbm, out1_hbm,
                 dst_v, hist_v, iota_v, acc):
  c = lax.axis_index("c")
  s = lax.axis_index("s")
  w = c * NS + s
  pltpu.sync_copy(dst_hbm.at[w], dst_v)
  pltpu.sync_copy(zeros_hbm, hist_v)
  pltpu.sync_copy(iota_hbm, iota_v)

  @pl.when(s == 0)
  def _():
    pltpu.sync_copy(hist_v, acc)
  ones16 = jnp.ones((16,), jnp.float32)

  def row(j, carry):
    for k in range(CHUNK // 16):
      d16 = dst_v[j, pl.ds(16 * k, 16)]
      plsc.addupdate_scatter(
          hist_v, [lax.shift_right_logical(d16, 7), lax.bitwise_and(d16, 127)],
          ones16)
    return carry

  lax.fori_loop(0, CPT, row, 0)
  plsc.subcore_barrier()
  pltpu.sync_copy(hist_v, acc.at[iota_v], add=True)
  plsc.subcore_barrier()
  @pl.when(s < HR // 8)
  def _():
    pltpu.sync_copy(acc.at[pl.ds(s * 8, 8)], hist_v.at[pl.ds(0, 8)])

    @pl.when(c == 0)
    def _():
      pltpu.sync_copy(hist_v.at[pl.ds(0, 8)], out0_hbm.at[pl.ds(s * 8, 8)])

    @pl.when(c == 1)
    def _():
      pltpu.sync_copy(hist_v.at[pl.ds(0, 8)], out1_hbm.at[pl.ds(s * 8, 8)])


def _sc_scatter_body(u_hbm, src_hbm, dst_hbm, zeros_hbm, out0_hbm, out1_hbm,
                     srcA, srcB, dstA, dstB, buf0, buf1,
                     sem0, sem1, semi, acc):
  c = lax.axis_index("c")
  s = lax.axis_index("s")
  w = c * NS + s
  row0 = s * ROWS_PER_TILE
  pltpu.sync_copy(src_hbm.at[w, pl.ds(0, SEC)], srcA)
  pltpu.sync_copy(dst_hbm.at[w, pl.ds(0, SEC)], dstA)
  pltpu.sync_copy(zeros_hbm, buf0)
  for z in range(ROWS_PER_TILE // CHUNK):
    pltpu.sync_copy(buf0, acc.at[pl.ds(row0 + z * CHUNK, CHUNK)])
  plsc.subcore_barrier()

  secs = [(srcA, dstA), (srcB, dstB)]
  pltpu.async_copy(u_hbm.at[srcA.at[0]], buf0, sem0)
  for sct in range(NSEC):
    src_v, dst_v = secs[sct % 2]
    nsrc_v, ndst_v = secs[(sct + 1) % 2]
    if sct + 1 < NSEC:
      ip0 = pltpu.async_copy(src_hbm.at[w, pl.ds((sct + 1) * SEC, SEC)],
                             nsrc_v, semi)
      ip1 = pltpu.async_copy(dst_hbm.at[w, pl.ds((sct + 1) * SEC, SEC)],
                             ndst_v, semi)

    def pair(j, carry, src_v=src_v, dst_v=dst_v):
      c0 = 2 * j
      pltpu.async_copy(u_hbm.at[src_v.at[c0 + 1]], buf1, sem1)
      pltpu.make_async_copy(u_hbm.at[src_v.at[c0]], buf0, sem0).wait()
      pltpu.sync_copy(buf0, acc.at[dst_v.at[c0]], add=True)

      @pl.when(j + 1 < SEC // 2)
      def _():
        pltpu.async_copy(u_hbm.at[src_v.at[c0 + 2]], buf0, sem0)

      pltpu.make_async_copy(u_hbm.at[src_v.at[c0 + 1]], buf1, sem1).wait()
      pltpu.sync_copy(buf1, acc.at[dst_v.at[c0 + 1]], add=True)
      return carry

    lax.fori_loop(0, SEC // 2, pair, 0)
    if sct + 1 < NSEC:
      ip0.wait()
      ip1.wait()
      pltpu.async_copy(u_hbm.at[nsrc_v.at[0]], buf0, sem0)

  plsc.subcore_barrier()
  for z in range(ROWS_PER_TILE // CHUNK):
    r = row0 + z * CHUNK
    pltpu.sync_copy(acc.at[pl.ds(r, CHUNK)], buf0)

    @pl.when(c == 0)
    def _():
      pltpu.sync_copy(buf0, out0_hbm.at[pl.ds(r, CHUNK)])

    @pl.when(c == 1)
    def _():
      pltpu.sync_copy(buf0, out1_hbm.at[pl.ds(r, CHUNK)])


def _tc1_body(xb, w1, d0b, d1b, ub, disb):
  dis8 = lax.rsqrt(d0b[...] + d1b[...] + 1.0)
  ii = lax.broadcasted_iota(jnp.int32, (D, D), 0)
  jj = lax.broadcasted_iota(jnp.int32, (D, D), 1)
  eye = jnp.where(ii == jj, 1.0, 0.0)
  ones_m = jnp.ones((D, D), jnp.float32)
  parts = [
      jnp.dot(eye * dis8[g:g + 1, :], ones_m,
              preferred_element_type=jnp.float32,
              precision=lax.Precision.HIGHEST)
      for g in range(BLK // D)
  ]
  dis_blk = jnp.concatenate(parts, axis=0)
  z = jnp.dot(xb[...], w1[...], preferred_element_type=jnp.float32)
  ub[...] = z * dis_blk
  disb[...] = dis_blk


def _tc2_body(q0b, q1b, u1b, w2, b1r, disb, ub):
  dis = disb[...]
  agg = (q0b[...] + q1b[...] + u1b[...]) * dis
  h = jnp.maximum(agg + b1r[...], 0.0)
  ub[...] = jnp.dot(h, w2[...], preferred_element_type=jnp.float32) * dis


def _tc3_body(q0b, q1b, u2b, b2r, disb, ob):
  ob[...] = (q0b[...] + q1b[...] + u2b[...]) * disb[...] + b2r[...]


def _node_specs(*widths):
  return [pl.BlockSpec((BLK, wd), lambda i: (i, 0)) for wd in widths]


def _full_spec(shape):
  return pl.BlockSpec(shape, lambda i: (0,) * len(shape))


@functools.lru_cache(maxsize=None)
def _sc_kernels():
  mesh = plsc.VectorSubcoreMesh(
      core_axis_name="c", subcore_axis_name="s", num_cores=NC, num_subcores=NS)
  sc_deg = pl.kernel(
      _sc_deg_body,
      out_type=(jax.ShapeDtypeStruct((HR, D), jnp.float32),
                jax.ShapeDtypeStruct((HR, D), jnp.float32)),
      mesh=mesh,
      scratch_types=[
          pltpu.VMEM((CPT, CHUNK), jnp.int32),
          pltpu.VMEM((HR, D), jnp.float32),
          pltpu.VMEM((HR,), jnp.int32),
          pltpu.VMEM_SHARED((HR, D), jnp.float32),
      ],
      compiler_params=pltpu.CompilerParams(needs_layout_passes=False),
  )
  sc_scatter = pl.kernel(
      _sc_scatter_body,
      out_type=(jax.ShapeDtypeStruct((NP, D), jnp.float32),
                jax.ShapeDtypeStruct((NP, D), jnp.float32)),
      mesh=mesh,
      scratch_types=[
          pltpu.VMEM((SEC, CHUNK), jnp.int32),
          pltpu.VMEM((SEC, CHUNK), jnp.int32),
          pltpu.VMEM((SEC, CHUNK), jnp.int32),
          pltpu.VMEM((SEC, CHUNK), jnp.int32),
          pltpu.VMEM((CHUNK, D), jnp.float32),
          pltpu.VMEM((CHUNK, D), jnp.float32),
          pltpu.SemaphoreType.DMA,
          pltpu.SemaphoreType.DMA,
          pltpu.SemaphoreType.DMA,
          pltpu.VMEM_SHARED((NP, D), jnp.float32),
      ],
  )
  return sc_deg, sc_scatter


_GRID = (NP // BLK,)

_DEG_SPEC = pl.BlockSpec((BLK // D, D), lambda i: (i, 0))

_tc1 = pl.pallas_call(
    _tc1_body,
    grid=_GRID,
    in_specs=[*_node_specs(D), _full_spec((D, D)), _DEG_SPEC, _DEG_SPEC],
    out_specs=[_node_specs(D)[0], _node_specs(D)[0]],
    out_shape=(jax.ShapeDtypeStruct((NP, D), jnp.float32),
               jax.ShapeDtypeStruct((NP, D), jnp.float32)),
)

_tc2 = pl.pallas_call(
    _tc2_body,
    grid=_GRID,
    in_specs=[*_node_specs(D, D, D), _full_spec((D, D)), _full_spec((1, D)),
              *_node_specs(D)],
    out_specs=_node_specs(D)[0],
    out_shape=jax.ShapeDtypeStruct((NP, D), jnp.float32),
)

_tc3 = pl.pallas_call(
    _tc3_body,
    grid=_GRID,
    in_specs=[*_node_specs(D, D, D), _full_spec((1, D)),
              *_node_specs(D)],
    out_specs=_node_specs(D)[0],
    out_shape=jax.ShapeDtypeStruct((N_REAL, D), jnp.float32),
)


@jax.jit
def kernel(x, edge_index, W1, b1, W2, b2):
  n_edges = edge_index.shape[1]
  src = edge_index[0].astype(jnp.int32)
  dst = edge_index[1].astype(jnp.int32)
  pad_n = E_PAD - n_edges
  pad_idx = N_REAL + (jnp.arange(pad_n, dtype=jnp.int32) % N_SCRATCH)
  src_p = jnp.concatenate([src, pad_idx]).reshape(NW, CPT, CHUNK)
  dst_p = jnp.concatenate([dst, pad_idx]).reshape(NW, CPT, CHUNK)

  zeros80 = jnp.zeros((HR, D), jnp.float32)
  iota80 = jnp.arange(HR, dtype=jnp.int32)
  zeros128 = jnp.zeros((CHUNK, D), jnp.float32)

  sc_deg, sc_scatter = _sc_kernels()
  d0, d1 = sc_deg(dst_p, zeros80, iota80)

  u1, disb = _tc1(x, W1, d0, d1)
  q0, q1 = sc_scatter(u1, src_p, dst_p, zeros128)
  u2 = _tc2(q0, q1, u1, W2, b1.reshape(1, D), disb)
  q20, q21 = sc_scatter(u2, src_p, dst_p, zeros128)
  return _tc3(q20, q21, u2, b2.reshape(1, D), disb)


# speedup vs baseline: 1.3263x; 1.0353x over previous
import functools

import jax
import jax.numpy as jnp
import numpy as np
from jax import lax
from jax.experimental import pallas as pl
from jax.experimental.pallas import tpu as pltpu
from jax.experimental.pallas import tpu_sc as plsc

NC = 2
NS = 16
NW = NC * NS
D = 128
N_REAL = 10000
NP = 10240
ROWS_PER_TILE = NP // NS
CHUNK = 128
SEC = 16
NSEC = 5
CPT = SEC * NSEC
E_PAD = NW * CPT * CHUNK
N_SCRATCH = NP - N_REAL
BLK = 1024


HR = NP // D


def _sc_deg_body(eidx_hbm, zeros_hbm, iota_hbm, out0_hbm, out1_hbm,
                 dst_v, hist_v, iota_v, acc):
  c = lax.axis_index("c")
  s = lax.axis_index("s")
  w = c * NS + s
  pltpu.sync_copy(eidx_hbm.at[1, pl.ds(w * CPT, CPT)], dst_v)
  pltpu.sync_copy(zeros_hbm, hist_v)
  pltpu.sync_copy(iota_hbm, iota_v)

  @pl.when(s == 0)
  def _():
    pltpu.sync_copy(hist_v, acc)
  ones16 = jnp.ones((16,), jnp.float32)

  def row(j, carry):
    for k in range(CHUNK // 16):
      d16 = dst_v[j, pl.ds(16 * k, 16)]
      plsc.addupdate_scatter(
          hist_v, [lax.shift_right_logical(d16, 7), lax.bitwise_and(d16, 127)],
          ones16)
    return carry

  lax.fori_loop(0, CPT, row, 0)
  plsc.subcore_barrier()
  pltpu.sync_copy(hist_v, acc.at[iota_v], add=True)
  plsc.subcore_barrier()
  @pl.when(s < HR // 8)
  def _():
    pltpu.sync_copy(acc.at[pl.ds(s * 8, 8)], hist_v.at[pl.ds(0, 8)])

    @pl.when(c == 0)
    def _():
      pltpu.sync_copy(hist_v.at[pl.ds(0, 8)], out0_hbm.at[pl.ds(s * 8, 8)])

    @pl.when(c == 1)
    def _():
      pltpu.sync_copy(hist_v.at[pl.ds(0, 8)], out1_hbm.at[pl.ds(s * 8, 8)])


def _sc_scatter_body(u_hbm, eidx_hbm, zeros_hbm, out0_hbm, out1_hbm,
                     srcA, srcB, dstA, dstB, buf0, buf1,
                     sem0, sem1, semi, acc):
  c = lax.axis_index("c")
  s = lax.axis_index("s")
  w = c * NS + s
  row0 = s * ROWS_PER_TILE
  pltpu.sync_copy(eidx_hbm.at[0, pl.ds(w * CPT, SEC)], srcA)
  pltpu.sync_copy(eidx_hbm.at[1, pl.ds(w * CPT, SEC)], dstA)
  pltpu.sync_copy(zeros_hbm, buf0)
  for z in range(ROWS_PER_TILE // CHUNK):
    pltpu.sync_copy(buf0, acc.at[pl.ds(row0 + z * CHUNK, CHUNK)])
  plsc.subcore_barrier()

  secs = [(srcA, dstA), (srcB, dstB)]
  pltpu.async_copy(u_hbm.at[srcA.at[0]], buf0, sem0)
  for sct in range(NSEC):
    src_v, dst_v = secs[sct % 2]
    nsrc_v, ndst_v = secs[(sct + 1) % 2]
    if sct + 1 < NSEC:
      ip0 = pltpu.async_copy(
          eidx_hbm.at[0, pl.ds(w * CPT + (sct + 1) * SEC, SEC)], nsrc_v, semi)
      ip1 = pltpu.async_copy(
          eidx_hbm.at[1, pl.ds(w * CPT + (sct + 1) * SEC, SEC)], ndst_v, semi)

    def pair(j, carry, src_v=src_v, dst_v=dst_v):
      c0 = 2 * j
      pltpu.async_copy(u_hbm.at[src_v.at[c0 + 1]], buf1, sem1)
      pltpu.make_async_copy(u_hbm.at[src_v.at[c0]], buf0, sem0).wait()
      pltpu.sync_copy(buf0, acc.at[dst_v.at[c0]], add=True)

      @pl.when(j + 1 < SEC // 2)
      def _():
        pltpu.async_copy(u_hbm.at[src_v.at[c0 + 2]], buf0, sem0)

      pltpu.make_async_copy(u_hbm.at[src_v.at[c0 + 1]], buf1, sem1).wait()
      pltpu.sync_copy(buf1, acc.at[dst_v.at[c0 + 1]], add=True)
      return carry

    lax.fori_loop(0, SEC // 2, pair, 0)
    if sct + 1 < NSEC:
      ip0.wait()
      ip1.wait()
      pltpu.async_copy(u_hbm.at[nsrc_v.at[0]], buf0, sem0)

  plsc.subcore_barrier()
  for z in range(ROWS_PER_TILE // CHUNK):
    r = row0 + z * CHUNK
    pltpu.sync_copy(acc.at[pl.ds(r, CHUNK)], buf0)

    @pl.when(c == 0)
    def _():
      pltpu.sync_copy(buf0, out0_hbm.at[pl.ds(r, CHUNK)])

    @pl.when(c == 1)
    def _():
      pltpu.sync_copy(buf0, out1_hbm.at[pl.ds(r, CHUNK)])


def _tcz_body(xb, w1, zb):
  zb[...] = jnp.dot(xb[...], w1[...], preferred_element_type=jnp.float32)


def _tcu_body(zb, d0b, d1b, ub, disb):
  dis8 = lax.rsqrt(d0b[...] + d1b[...] + 1.0)
  ii = lax.broadcasted_iota(jnp.int32, (D, D), 0)
  jj = lax.broadcasted_iota(jnp.int32, (D, D), 1)
  eye = jnp.where(ii == jj, 1.0, 0.0)
  ones_m = jnp.ones((D, D), jnp.float32)
  parts = [
      jnp.dot(eye * dis8[g:g + 1, :], ones_m,
              preferred_element_type=jnp.float32,
              precision=lax.Precision.HIGHEST)
      for g in range(BLK // D)
  ]
  dis_blk = jnp.concatenate(parts, axis=0)
  ub[...] = zb[...] * dis_blk
  disb[...] = dis_blk


def _tc2_body(q0b, q1b, u1b, w2, b1r, disb, ub):
  dis = disb[...]
  agg = (q0b[...] + q1b[...] + u1b[...]) * dis
  h = jnp.maximum(agg + b1r[...], 0.0)
  ub[...] = jnp.dot(h, w2[...], preferred_element_type=jnp.float32) * dis


def _tc3_body(q0b, q1b, u2b, b2r, disb, ob):
  ob[...] = (q0b[...] + q1b[...] + u2b[...]) * disb[...] + b2r[...]


def _node_specs(*widths):
  return [pl.BlockSpec((BLK, wd), lambda i: (i, 0)) for wd in widths]


def _full_spec(shape):
  return pl.BlockSpec(shape, lambda i: (0,) * len(shape))


@functools.lru_cache(maxsize=None)
def _sc_kernels():
  mesh = plsc.VectorSubcoreMesh(
      core_axis_name="c", subcore_axis_name="s", num_cores=NC, num_subcores=NS)
  sc_deg = pl.kernel(
      _sc_deg_body,
      out_type=(jax.ShapeDtypeStruct((HR, D), jnp.float32),
                jax.ShapeDtypeStruct((HR, D), jnp.float32)),
      mesh=mesh,
      scratch_types=[
          pltpu.VMEM((CPT, CHUNK), jnp.int32),
          pltpu.VMEM((HR, D), jnp.float32),
          pltpu.VMEM((HR,), jnp.int32),
          pltpu.VMEM_SHARED((HR, D), jnp.float32),
      ],
      compiler_params=pltpu.CompilerParams(needs_layout_passes=False),
  )
  sc_scatter = pl.kernel(
      _sc_scatter_body,
      out_type=(jax.ShapeDtypeStruct((NP, D), jnp.float32),
                jax.ShapeDtypeStruct((NP, D), jnp.float32)),
      mesh=mesh,
      scratch_types=[
          pltpu.VMEM((SEC, CHUNK), jnp.int32),
          pltpu.VMEM((SEC, CHUNK), jnp.int32),
          pltpu.VMEM((SEC, CHUNK), jnp.int32),
          pltpu.VMEM((SEC, CHUNK), jnp.int32),
          pltpu.VMEM((CHUNK, D), jnp.float32),
          pltpu.VMEM((CHUNK, D), jnp.float32),
          pltpu.SemaphoreType.DMA,
          pltpu.SemaphoreType.DMA,
          pltpu.SemaphoreType.DMA,
          pltpu.VMEM_SHARED((NP, D), jnp.float32),
      ],
  )
  return sc_deg, sc_scatter


_GRID = (NP // BLK,)

_DEG_SPEC = pl.BlockSpec((BLK // D, D), lambda i: (i, 0))

_tcz = pl.pallas_call(
    _tcz_body,
    grid=_GRID,
    in_specs=[*_node_specs(D), _full_spec((D, D))],
    out_specs=_node_specs(D)[0],
    out_shape=jax.ShapeDtypeStruct((NP, D), jnp.float32),
)

_tcu = pl.pallas_call(
    _tcu_body,
    grid=_GRID,
    in_specs=[*_node_specs(D), _DEG_SPEC, _DEG_SPEC],
    out_specs=[_node_specs(D)[0], _node_specs(D)[0]],
    out_shape=(jax.ShapeDtypeStruct((NP, D), jnp.float32),
               jax.ShapeDtypeStruct((NP, D), jnp.float32)),
)

_tc2 = pl.pallas_call(
    _tc2_body,
    grid=_GRID,
    in_specs=[*_node_specs(D, D, D), _full_spec((D, D)), _full_spec((1, D)),
              *_node_specs(D)],
    out_specs=_node_specs(D)[0],
    out_shape=jax.ShapeDtypeStruct((NP, D), jnp.float32),
)

_tc3 = pl.pallas_call(
    _tc3_body,
    grid=_GRID,
    in_specs=[*_node_specs(D, D, D), _full_spec((1, D)),
              *_node_specs(D)],
    out_specs=_node_specs(D)[0],
    out_shape=jax.ShapeDtypeStruct((N_REAL, D), jnp.float32),
)


_N_CHUNK_REAL = 320000 // CHUNK
_PAD_CHUNKS = E_PAD // CHUNK - _N_CHUNK_REAL
_PAD_BLOCK = np.asarray(
    N_REAL + np.arange(_PAD_CHUNKS * CHUNK) % N_SCRATCH,
    np.int32).reshape(1, _PAD_CHUNKS, CHUNK).repeat(2, axis=0)


@jax.jit
def kernel(x, edge_index, W1, b1, W2, b2):
  ei = edge_index.astype(jnp.int32).reshape(2, _N_CHUNK_REAL, CHUNK)
  eidx = jnp.concatenate([ei, jnp.asarray(_PAD_BLOCK)], axis=1)

  zeros80 = jnp.zeros((HR, D), jnp.float32)
  iota80 = jnp.arange(HR, dtype=jnp.int32)
  zeros128 = jnp.zeros((CHUNK, D), jnp.float32)

  sc_deg, sc_scatter = _sc_kernels()
  d0, d1 = sc_deg(eidx, zeros80, iota80)

  z1 = _tcz(x, W1)
  u1, disb = _tcu(z1, d0, d1)
  q0, q1 = sc_scatter(u1, eidx, zeros128)
  u2 = _tc2(q0, q1, u1, W2, b1.reshape(1, D), disb)
  q20, q21 = sc_scatter(u2, eidx, zeros128)
  return _tc3(q20, q21, u2, b2.reshape(1, D), disb)
